# Initial kernel scaffold; baseline (speedup 1.0000x reference)
import jax
import jax.numpy as jnp
from jax.experimental import pallas as pl


def kernel(x, Wq, bq, Wk, bk, Wv, bv, Wo, bo, ln1_g, ln1_b, ln2_g, ln2_b, Wg, W1, b1, W2, b2):
    raise NotImplementedError("write your pallas kernel here")


# speedup vs baseline: 2.0050x; 2.0050x over previous
import functools

import numpy as np
import jax
import jax.numpy as jnp
from jax import lax
from jax.experimental import pallas as pl
from jax.experimental.pallas import tpu as pltpu
from jax.experimental.pallas import tpu_sc as plsc

S, B, D = 2048, 1, 1024
H = 16
DH = D // H
E = 8
TOPK = 2
DFF = 4096
T = S * B
CAP = TOPK * T // E
EP = 128
NW = 32
CH = 64
EPW = T * TOPK // NW
BUF_ROWS = E * CAP + NW

_PC = pl.pallas_call

_TRI = jnp.asarray(np.tril(np.ones((T, T), np.float32), -1), jnp.bfloat16)


def _ln(x, g, b):
    m = jnp.mean(x, axis=1, keepdims=True)
    c = x - m
    var = jnp.mean(c * c, axis=1, keepdims=True)
    return c * lax.rsqrt(var + 1e-5) * g + b


def _qkv_body(x_ref, wq_ref, wk_ref, wv_ref, bq_ref, bk_ref, bv_ref,
              g_ref, b_ref, q_ref, k_ref, v_ref):
    h = _ln(x_ref[...], g_ref[...], b_ref[...])
    q = jnp.dot(h, wq_ref[...], preferred_element_type=jnp.float32) + bq_ref[...]
    q_ref[...] = q * (DH ** -0.5)
    k_ref[...] = jnp.dot(h, wk_ref[...], preferred_element_type=jnp.float32) + bk_ref[...]
    v_ref[...] = jnp.dot(h, wv_ref[...], preferred_element_type=jnp.float32) + bv_ref[...]


def _attn_body(q_ref, k_ref, v_ref, o_ref):
    outs = []
    for j in range(2):
        sl = slice(j * DH, (j + 1) * DH)
        q = q_ref[:, sl]
        k = k_ref[:, sl]
        s = lax.dot_general(q, k, (((1,), (1,)), ((), ())),
                            preferred_element_type=jnp.float32)
        mx = jnp.max(s, axis=1, keepdims=True)
        p = jnp.exp(s - mx)
        p = p / jnp.sum(p, axis=1, keepdims=True)
        outs.append(jnp.dot(p, v_ref[:, sl], preferred_element_type=jnp.float32))
    o_ref[...] = jnp.concatenate(outs, axis=1)


def _proj_body(x_ref, o_ref, wo_ref, bo_ref, y_ref):
    y_ref[...] = (x_ref[...]
                  + jnp.dot(o_ref[...], wo_ref[...], preferred_element_type=jnp.float32)
                  + bo_ref[...])


def _route_body(x_ref, g_ref, b_ref, wg_ref, tri_ref,
                h2_ref, idx_ref, w_ref):
    h2 = _ln(x_ref[...], g_ref[...], b_ref[...])
    h2_ref[...] = h2
    logits = jnp.dot(h2, wg_ref[...], preferred_element_type=jnp.float32)
    col = lax.broadcasted_iota(jnp.int32, (T, EP), 1)
    valid = col < E
    lg = jnp.where(valid, logits, -1e30)
    mx = jnp.max(lg, axis=1, keepdims=True)
    ex = jnp.where(valid, jnp.exp(lg - mx), 0.0)
    gates = ex / jnp.sum(ex, axis=1, keepdims=True)
    v0 = jnp.max(gates, axis=1, keepdims=True)
    i0 = jnp.min(jnp.where(gates == v0, col, EP), axis=1, keepdims=True)
    g1 = jnp.where(col == i0, -1.0, gates)
    v1 = jnp.max(g1, axis=1, keepdims=True)
    i1 = jnp.min(jnp.where(g1 == v1, col, EP), axis=1, keepdims=True)
    m0 = (col == i0).astype(jnp.bfloat16)
    m1 = (col == i1).astype(jnp.bfloat16)
    tri = tri_ref[...]
    c0 = jnp.dot(tri, m0, preferred_element_type=jnp.float32)
    tot0 = jnp.sum(m0.astype(jnp.float32), axis=0, keepdims=True)
    c1 = jnp.dot(tri, m1, preferred_element_type=jnp.float32) + tot0
    pos0 = jnp.sum(jnp.where(col == i0, c0, 0.0), axis=1, keepdims=True)
    pos1 = jnp.sum(jnp.where(col == i1, c1, 0.0), axis=1, keepdims=True)
    keep0 = pos0 < CAP
    keep1 = pos1 < CAP
    denom = v0 + v1 + 1e-9
    w0 = jnp.where(keep0, v0 / denom, 0.0)
    w1 = jnp.where(keep1, v1 / denom, 0.0)
    p0 = pos0.astype(jnp.int32)
    p1 = pos1.astype(jnp.int32)
    tok = lax.broadcasted_iota(jnp.int32, (T, 1), 0)
    d0 = E * CAP + tok // EPW
    d1 = E * CAP + (NW // 2) + tok // EPW
    s0 = jnp.where(keep0, i0 * CAP + p0, d0)
    s1 = jnp.where(keep1, i1 * CAP + p1, d1)
    gi0 = jnp.where(keep0, i0 * CAP + p0, 0)
    gi1 = jnp.where(keep1, i1 * CAP + p1, 0)
    idx_ref[...] = jnp.concatenate([s0, s1, gi0, gi1, s0, s0, s0, s0], axis=1)
    w_ref[...] = jnp.concatenate([w0, w1, w0, w0, w0, w0, w0, w0], axis=1)


def _ffn_body(buf_ref, w1_ref, b1_ref, w2_ref, b2_ref, out_ref):
    c = pl.program_id(1)
    h = jnp.maximum(
        jnp.dot(buf_ref[...], w1_ref[0], preferred_element_type=jnp.float32)
        + b1_ref[0], 0.0)
    contrib = jnp.dot(h, w2_ref[0], preferred_element_type=jnp.float32)

    @pl.when(c == 0)
    def _():
        out_ref[...] = contrib + b2_ref[0]

    @pl.when(c != 0)
    def _():
        out_ref[...] += contrib


def _combine_body(x_ref, g0_ref, g1_ref, w_ref, o_ref):
    w0 = w_ref[:, 0:1]
    w1 = w_ref[:, 1:2]
    a0 = jnp.where(w0 > 0, w0 * g0_ref[...], 0.0)
    a1 = jnp.where(w1 > 0, w1 * g1_ref[...], 0.0)
    o_ref[...] = x_ref[...] + a0 + a1


def _dispatch(tokens, sidx):
    mesh = plsc.VectorSubcoreMesh(core_axis_name="c", subcore_axis_name="s")

    @functools.partial(
        pl.kernel, mesh=mesh,
        out_type=jax.ShapeDtypeStruct((BUF_ROWS, D), jnp.float32),
        scratch_types=[pltpu.VMEM((CH,), jnp.int32),
                       pltpu.VMEM((CH, D), jnp.float32),
                       pltpu.SemaphoreType.DMA])
    def k(tok_hbm, idx_hbm, buf_hbm, idx_v, rows_v, sem):
        wid = lax.axis_index("s") * 2 + lax.axis_index("c")
        base = wid * EPW
        for cnk in range(EPW // CH):
            off = base + cnk * CH
            pltpu.sync_copy(idx_hbm.at[pl.ds(off, CH)], idx_v)
            pltpu.sync_copy(tok_hbm.at[pl.ds(lax.rem(off, T), CH)], rows_v)
            pltpu.async_copy(rows_v, buf_hbm.at[idx_v], sem).wait()

    return k(tokens, sidx)


def _gather_rows(eout, gidx):
    mesh = plsc.VectorSubcoreMesh(core_axis_name="c", subcore_axis_name="s")

    @functools.partial(
        pl.kernel, mesh=mesh,
        out_type=jax.ShapeDtypeStruct((T * TOPK, D), jnp.float32),
        scratch_types=[pltpu.VMEM((CH,), jnp.int32),
                       pltpu.VMEM((CH, D), jnp.float32),
                       pltpu.SemaphoreType.DMA])
    def k(e_hbm, idx_hbm, out_hbm, idx_v, rows_v, sem):
        wid = lax.axis_index("s") * 2 + lax.axis_index("c")
        base = wid * EPW
        for cnk in range(EPW // CH):
            off = base + cnk * CH
            pltpu.sync_copy(idx_hbm.at[pl.ds(off, CH)], idx_v)
            pltpu.async_copy(e_hbm.at[idx_v], rows_v, sem).wait()
            pltpu.sync_copy(rows_v, out_hbm.at[pl.ds(off, CH)])

    return k(eout, gidx)


def kernel(x, Wq, bq, Wk, bk, Wv, bv, Wo, bo, ln1_g, ln1_b, ln2_g, ln2_b,
           Wg, W1, b1, W2, b2):
    xt = x.reshape(T, D)
    r2 = lambda a: a.reshape(1, -1)
    RB = 256

    f32 = jnp.float32
    q, k, v = _PC(
        _qkv_body,
        grid=(T // RB,),
        in_specs=[
            pl.BlockSpec((RB, D), lambda i: (i, 0)),
            pl.BlockSpec((D, D), lambda i: (0, 0)),
            pl.BlockSpec((D, D), lambda i: (0, 0)),
            pl.BlockSpec((D, D), lambda i: (0, 0)),
            pl.BlockSpec((1, D), lambda i: (0, 0)),
            pl.BlockSpec((1, D), lambda i: (0, 0)),
            pl.BlockSpec((1, D), lambda i: (0, 0)),
            pl.BlockSpec((1, D), lambda i: (0, 0)),
            pl.BlockSpec((1, D), lambda i: (0, 0)),
        ],
        out_specs=[pl.BlockSpec((RB, D), lambda i: (i, 0))] * 3,
        out_shape=[jax.ShapeDtypeStruct((T, D), f32)] * 3,
    )(xt, Wq, Wk, Wv, r2(bq), r2(bk), r2(bv), r2(ln1_g), r2(ln1_b))

    o = _PC(
        _attn_body,
        grid=(H // 2,),
        in_specs=[pl.BlockSpec((T, 2 * DH), lambda h: (0, h))] * 3,
        out_specs=pl.BlockSpec((T, 2 * DH), lambda h: (0, h)),
        out_shape=jax.ShapeDtypeStruct((T, D), f32),
    )(q, k, v)

    x2 = _PC(
        _proj_body,
        grid=(T // RB,),
        in_specs=[
            pl.BlockSpec((RB, D), lambda i: (i, 0)),
            pl.BlockSpec((RB, D), lambda i: (i, 0)),
            pl.BlockSpec((D, D), lambda i: (0, 0)),
            pl.BlockSpec((1, D), lambda i: (0, 0)),
        ],
        out_specs=pl.BlockSpec((RB, D), lambda i: (i, 0)),
        out_shape=jax.ShapeDtypeStruct((T, D), f32),
    )(xt, o, Wo, r2(bo))

    wg_pad = jnp.pad(Wg, ((0, 0), (0, EP - E)))
    h2, idx, w = _PC(
        _route_body,
        in_specs=[pl.BlockSpec(x2.shape, lambda: (0, 0)),
                  pl.BlockSpec((1, D), lambda: (0, 0)),
                  pl.BlockSpec((1, D), lambda: (0, 0)),
                  pl.BlockSpec((D, EP), lambda: (0, 0)),
                  pl.BlockSpec((T, T), lambda: (0, 0))],
        out_specs=[pl.BlockSpec((T, D), lambda: (0, 0)),
                   pl.BlockSpec((T, 8), lambda: (0, 0)),
                   pl.BlockSpec((T, 8), lambda: (0, 0))],
        out_shape=[jax.ShapeDtypeStruct((T, D), f32),
                   jax.ShapeDtypeStruct((T, 8), jnp.int32),
                   jax.ShapeDtypeStruct((T, 8), f32)],
    )(x2, r2(ln2_g), r2(ln2_b), wg_pad, _TRI)

    sidx = jnp.concatenate([idx[:, 0], idx[:, 1]])
    gidx = jnp.concatenate([idx[:, 2], idx[:, 3]])

    buf = _dispatch(h2, sidx)

    DC = 1024
    eout = _PC(
        _ffn_body,
        grid=(E, DFF // DC),
        in_specs=[
            pl.BlockSpec((CAP, D), lambda e, c: (e, 0)),
            pl.BlockSpec((1, D, DC), lambda e, c: (e, 0, c)),
            pl.BlockSpec((1, 1, DC), lambda e, c: (e, 0, c)),
            pl.BlockSpec((1, DC, D), lambda e, c: (e, c, 0)),
            pl.BlockSpec((1, 1, D), lambda e, c: (e, 0, 0)),
        ],
        out_specs=pl.BlockSpec((CAP, D), lambda e, c: (e, 0)),
        out_shape=jax.ShapeDtypeStruct((E * CAP, D), f32),
    )(buf, W1, b1.reshape(E, 1, DFF), W2, b2.reshape(E, 1, D))

    gath = _gather_rows(eout, gidx)

    out = _PC(
        _combine_body,
        grid=(T // RB,),
        in_specs=[
            pl.BlockSpec((RB, D), lambda i: (i, 0)),
            pl.BlockSpec((RB, D), lambda i: (i, 0)),
            pl.BlockSpec((RB, D), lambda i: (i + T // RB, 0)),
            pl.BlockSpec((RB, 8), lambda i: (i, 0)),
        ],
        out_specs=pl.BlockSpec((RB, D), lambda i: (i, 0)),
        out_shape=jax.ShapeDtypeStruct((T, D), f32),
    )(x2, gath, gath, w)

    return out.reshape(S, B, D)
